# baseline (speedup 1.0000x reference)
import functools

import jax
import jax.numpy as jnp
import numpy as np
from jax import lax
from jax.experimental import pallas as pl
from jax.experimental.pallas import tpu as pltpu
from jax.experimental.pallas import tpu_sc as plsc

B, A = 128, 128
L3 = 3 * A
BI = 128


NB = 8


def _dense_body(post_ref, counts_ref, dist_ref, vec_ref):
    for bb in range(NB):
        xT = post_ref[bb]
        posi = jnp.transpose(xT)
        counts = counts_ref[bb]
        dx = jnp.broadcast_to(xT[0:1, :], (BI, A)) - posi[:, 0:1]
        dy = jnp.broadcast_to(xT[1:2, :], (BI, A)) - posi[:, 1:2]
        dz = jnp.broadcast_to(xT[2:3, :], (BI, A)) - posi[:, 2:3]
        d2 = dx * dx + dy * dy + dz * dz
        dist_ref[bb] = counts * jnp.sqrt(d2)
        vec_ref[bb, 0] = counts * dx
        vec_ref[bb, 1] = counts * dy
        vec_ref[bb, 2] = counts * dz


def _dense_stage(positions, counts):
    post = positions.transpose(0, 2, 1)
    grid = (B // NB,)
    dist, vecp = pl.pallas_call(
        _dense_body,
        grid=grid,
        in_specs=[
            pl.BlockSpec((NB, 3, A), lambda b: (b, 0, 0)),
            pl.BlockSpec((NB, BI, A), lambda b: (b, 0, 0)),
        ],
        out_specs=[
            pl.BlockSpec((NB, BI, A), lambda b: (b, 0, 0)),
            pl.BlockSpec((NB, 3, BI, A), lambda b: (b, 0, 0, 0)),
        ],
        out_shape=[
            jax.ShapeDtypeStruct((B, A, A), jnp.float32),
            jax.ShapeDtypeStruct((B, 3, A, A), jnp.float32),
        ],
    )(post, counts)
    return dist, vecp.transpose(0, 2, 3, 1)


E = 524288
NC, NS = 2, 16
HALF = B * A * A // NC
EPT = E // NS
CH = 8192
CHR = CH // 128
ZB = 2048
ZSEG = HALF // NS


def _counts_body(nm_hbm, out_hbm, bbuf, ibuf, jbuf, idx2, ones, zbuf, shared):
    c = lax.axis_index("c")
    s = lax.axis_index("s")
    base = c * HALF

    def _fill(k, _):
        zbuf[pl.ds(k * 16, 16)] = jnp.zeros((16,), jnp.float32)
        return _
    lax.fori_loop(0, ZB // 16, _fill, 0)

    def _fill1(k, _):
        ones[pl.ds(k * 16, 16)] = jnp.ones((16,), jnp.float32)
        return _
    lax.fori_loop(0, 8, _fill1, 0)

    def _zcopy(k, _):
        pltpu.sync_copy(zbuf, shared.at[pl.ds(s * ZSEG + k * ZB, ZB)])
        return _
    lax.fori_loop(0, ZSEG // ZB, _zcopy, 0)
    plsc.subcore_barrier()

    for ch in range(EPT // CH):
        off = s * EPT + ch * CH
        pltpu.sync_copy(nm_hbm.at[pl.ds(0, 1), pl.ds(off, CH)], bbuf)
        pltpu.sync_copy(nm_hbm.at[pl.ds(1, 1), pl.ds(off, CH)], ibuf)
        pltpu.sync_copy(nm_hbm.at[pl.ds(2, 1), pl.ds(off, CH)], jbuf)

        def _row(r, _):
            for q in range(8):
                sl = pl.ds(r * 128 + q * 16, 16)
                flat = bbuf[0, sl] * (A * A) + ibuf[0, sl] * A + jbuf[0, sl]
                loc = flat - base
                bad = (loc < 0) | (loc >= HALF)
                loc = jnp.where(bad, HALF + (flat & 127), loc)
                idx2[r, pl.ds(q * 16, 16)] = loc
            return _
        lax.fori_loop(0, CHR, _row, 0)

        def _srow(r, _):
            pltpu.sync_copy(ones, shared.at[idx2.at[r]], add=True)
            return _
        lax.fori_loop(0, CHR, _srow, 0)
    plsc.subcore_barrier()

    pltpu.sync_copy(shared.at[pl.ds(s * ZSEG, ZSEG)],
                    out_hbm.at[pl.ds(base + s * ZSEG, ZSEG)])


def _counts_stage(neighbor_mask):
    f = pl.kernel(
        _counts_body,
        out_type=jax.ShapeDtypeStruct((B * A * A,), jnp.float32),
        mesh=plsc.VectorSubcoreMesh(core_axis_name="c", subcore_axis_name="s"),
        scratch_types=[
            pltpu.VMEM((1, CH), jnp.int32),
            pltpu.VMEM((1, CH), jnp.int32),
            pltpu.VMEM((1, CH), jnp.int32),
            pltpu.VMEM((CHR, 128), jnp.int32),
            pltpu.VMEM((128,), jnp.float32),
            pltpu.VMEM((ZB,), jnp.float32),
            pltpu.VMEM_SHARED((HALF + 128,), jnp.float32),
        ],
    )
    return f(neighbor_mask)


def kernel(positions, neighbor_mask):
    counts = _counts_stage(neighbor_mask).reshape(B, A, A)
    return _dense_stage(positions, counts)


# speedup vs baseline: 1.1846x; 1.1846x over previous
import functools

import jax
import jax.numpy as jnp
import numpy as np
from jax import lax
from jax.experimental import pallas as pl
from jax.experimental.pallas import tpu as pltpu
from jax.experimental.pallas import tpu_sc as plsc

B, A = 128, 128
L3 = 3 * A
BI = 128


NB = 8


def _dense_body(post_ref, counts_ref, dist_ref, vec_ref):
    for bb in range(NB):
        xT = post_ref[bb]
        posi = jnp.transpose(xT)
        counts = counts_ref[bb]
        dx = jnp.broadcast_to(xT[0:1, :], (BI, A)) - posi[:, 0:1]
        dy = jnp.broadcast_to(xT[1:2, :], (BI, A)) - posi[:, 1:2]
        dz = jnp.broadcast_to(xT[2:3, :], (BI, A)) - posi[:, 2:3]
        d2 = dx * dx + dy * dy + dz * dz
        dist_ref[bb] = counts * jnp.sqrt(d2)
        vec_ref[bb, 0] = counts * dx
        vec_ref[bb, 1] = counts * dy
        vec_ref[bb, 2] = counts * dz


def _dense_stage(positions, counts):
    post = positions.transpose(0, 2, 1)
    grid = (B // NB,)
    dist, vecp = pl.pallas_call(
        _dense_body,
        grid=grid,
        in_specs=[
            pl.BlockSpec((NB, 3, A), lambda b: (b, 0, 0)),
            pl.BlockSpec((NB, BI, A), lambda b: (b, 0, 0)),
        ],
        out_specs=[
            pl.BlockSpec((NB, BI, A), lambda b: (b, 0, 0)),
            pl.BlockSpec((NB, 3, BI, A), lambda b: (b, 0, 0, 0)),
        ],
        out_shape=[
            jax.ShapeDtypeStruct((B, A, A), jnp.float32),
            jax.ShapeDtypeStruct((B, 3, A, A), jnp.float32),
        ],
    )(post, counts)
    return dist, vecp.transpose(0, 2, 3, 1)


E = 524288
NC, NS = 2, 16
HALF = B * A * A // NC
EPT = E // NS
NCHUNK = 4
CH = EPT // NCHUNK
CHR = CH // 128
TROWS = EPT // 128
TPAD = 8192
ZB = 2048
ZSEG = HALF // NS


def _counts_body(nm_hbm, out_hbm, bbuf, ibuf, jbuf, idx2, ones, zbuf, shared,
                 sem0, sem1):
    c = lax.axis_index("c")
    s = lax.axis_index("s")
    base = c * HALF
    sems = (sem0, sem1)

    def _fill(k, _):
        zbuf[pl.ds(k * 16, 16)] = jnp.zeros((16,), jnp.float32)
        return _
    lax.fori_loop(0, ZB // 16, _fill, 0)

    def _fill1(k, _):
        ones[pl.ds(k * 16, 16)] = jnp.ones((16,), jnp.float32)
        return _
    lax.fori_loop(0, 8, _fill1, 0)

    def _zcopy(k, _):
        pltpu.sync_copy(zbuf, shared.at[pl.ds(s * ZSEG + k * ZB, ZB)])
        return _
    lax.fori_loop(0, ZSEG // ZB, _zcopy, 0)
    plsc.subcore_barrier()

    pending = []
    for ch in range(NCHUNK):
        off = s * EPT + ch * CH
        pltpu.sync_copy(nm_hbm.at[pl.ds(0, 1), pl.ds(off, CH)], bbuf)
        pltpu.sync_copy(nm_hbm.at[pl.ds(1, 1), pl.ds(off, CH)], ibuf)
        pltpu.sync_copy(nm_hbm.at[pl.ds(2, 1), pl.ds(off, CH)], jbuf)

        def _row(r, _):
            for q in range(8):
                sl = pl.ds(r * 128 + q * 16, 16)
                flat = bbuf[0, sl] * (A * A) + ibuf[0, sl] * A + jbuf[0, sl]
                loc = flat - base
                bad = (loc < 0) | (loc >= HALF)
                loc = jnp.where(bad, HALF + (flat & (TPAD - 1)), loc)
                idx2[ch * CHR + r, pl.ds(q * 16, 16)] = loc
            return _
        lax.fori_loop(0, CHR, _row, 0)

        for h in pending:
            h.wait()
        pending = [
            pltpu.async_copy(ones, shared.at[idx2.at[ch * CHR + r]],
                             sems[ch % 2], add=True)
            for r in range(CHR)
        ]
    for h in pending:
        h.wait()
    plsc.subcore_barrier()

    pltpu.sync_copy(shared.at[pl.ds(s * ZSEG, ZSEG)],
                    out_hbm.at[pl.ds(base + s * ZSEG, ZSEG)])


def _counts_stage(neighbor_mask):
    f = pl.kernel(
        _counts_body,
        out_type=jax.ShapeDtypeStruct((B * A * A,), jnp.float32),
        mesh=plsc.VectorSubcoreMesh(core_axis_name="c", subcore_axis_name="s"),
        scratch_types=[
            pltpu.VMEM((1, CH), jnp.int32),
            pltpu.VMEM((1, CH), jnp.int32),
            pltpu.VMEM((1, CH), jnp.int32),
            pltpu.VMEM((TROWS, 128), jnp.int32),
            pltpu.VMEM((128,), jnp.float32),
            pltpu.VMEM((ZB,), jnp.float32),
            pltpu.VMEM_SHARED((HALF + TPAD,), jnp.float32),
            pltpu.SemaphoreType.DMA,
            pltpu.SemaphoreType.DMA,
        ],
    )
    return f(neighbor_mask)


def kernel(positions, neighbor_mask):
    counts = _counts_stage(neighbor_mask).reshape(B, A, A)
    return _dense_stage(positions, counts)


# speedup vs baseline: 1.3966x; 1.1790x over previous
import functools

import jax
import jax.numpy as jnp
import numpy as np
from jax import lax
from jax.experimental import pallas as pl
from jax.experimental.pallas import tpu as pltpu
from jax.experimental.pallas import tpu_sc as plsc

B, A = 128, 128
L3 = 3 * A
BI = 128


NB = 8


def _dense_body(post_ref, counts_ref, dist_ref, vec_ref):
    for bb in range(NB):
        xT = post_ref[bb]
        posi = jnp.transpose(xT)
        counts = counts_ref[bb]
        dx = jnp.broadcast_to(xT[0:1, :], (BI, A)) - posi[:, 0:1]
        dy = jnp.broadcast_to(xT[1:2, :], (BI, A)) - posi[:, 1:2]
        dz = jnp.broadcast_to(xT[2:3, :], (BI, A)) - posi[:, 2:3]
        d2 = dx * dx + dy * dy + dz * dz
        dist_ref[bb] = counts * jnp.sqrt(d2)
        vec_ref[bb, 0] = counts * dx
        vec_ref[bb, 1] = counts * dy
        vec_ref[bb, 2] = counts * dz


def _dense_stage(positions, counts):
    post = positions.transpose(0, 2, 1)
    grid = (B // NB,)
    dist, vecp = pl.pallas_call(
        _dense_body,
        grid=grid,
        in_specs=[
            pl.BlockSpec((NB, 3, A), lambda b: (b, 0, 0)),
            pl.BlockSpec((NB, BI, A), lambda b: (b, 0, 0)),
        ],
        out_specs=[
            pl.BlockSpec((NB, BI, A), lambda b: (b, 0, 0)),
            pl.BlockSpec((NB, 3, BI, A), lambda b: (b, 0, 0, 0)),
        ],
        out_shape=[
            jax.ShapeDtypeStruct((B, A, A), jnp.float32),
            jax.ShapeDtypeStruct((B, 3, A, A), jnp.float32),
        ],
    )(post, counts)
    return dist, vecp.transpose(0, 2, 3, 1)


E = 524288
NC, NS = 2, 16
HALF = B * A * A // NC
EPT = E // NS
NCHUNK = 8
CH = EPT // NCHUNK
CHR = CH // 128
TROWS = EPT // 128
TPAD = 8192
ZB = 2048
ZSEG = HALF // NS


def _counts_body(nm_hbm, out_hbm, b0, i0, j0, b1, i1, j1, idx2, ones, zbuf,
                 shared, sem0, sem1, stg0, stg1):
    c = lax.axis_index("c")
    s = lax.axis_index("s")
    sems = (sem0, sem1)
    stgs = (stg0, stg1)
    bufs = ((b0, i0, j0), (b1, i1, j1))

    def _fill(k, _):
        zbuf[pl.ds(k * 16, 16)] = jnp.zeros((16,), jnp.float32)
        return _
    lax.fori_loop(0, ZB // 16, _fill, 0)

    def _fill1(k, _):
        ones[pl.ds(k * 16, 16)] = jnp.ones((16,), jnp.float32)
        return _
    lax.fori_loop(0, 8, _fill1, 0)

    def _stage(ch):
        off = s * EPT + ch * CH
        trip = bufs[ch % 2]
        sem = stgs[ch % 2]
        return [pltpu.async_copy(nm_hbm.at[pl.ds(r, 1), pl.ds(off, CH)],
                                 trip[r], sem) for r in range(3)]
    pend_stage = _stage(0)

    def _zcopy(k, _):
        pltpu.sync_copy(zbuf, shared.at[pl.ds(s * ZSEG + k * ZB, ZB)])
        return _
    lax.fori_loop(0, ZSEG // ZB, _zcopy, 0)
    plsc.subcore_barrier()

    pend_scatter = []
    for ch in range(NCHUNK):
        nxt = _stage(ch + 1) if ch + 1 < NCHUNK else []
        for h in pend_stage:
            h.wait()
        bb, ib, jb = bufs[ch % 2]

        def _row(r, _):
            for q in range(8):
                sl = pl.ds(r * 128 + q * 16, 16)
                fl = (bb[0, sl] << 14) | (ib[0, sl] << 7) | jb[0, sl]
                bad = (fl >> 20) != c
                loc = jnp.where(bad, (fl & (TPAD - 1)) | HALF, fl & (HALF - 1))
                idx2[ch * CHR + r, pl.ds(q * 16, 16)] = loc
            return _
        lax.fori_loop(0, CHR, _row, 0)

        for h in pend_scatter:
            h.wait()
        pend_scatter = [
            pltpu.async_copy(ones, shared.at[idx2.at[ch * CHR + r]],
                             sems[ch % 2], add=True)
            for r in range(CHR)
        ]
        pend_stage = nxt
    for h in pend_scatter:
        h.wait()
    plsc.subcore_barrier()

    pltpu.sync_copy(shared.at[pl.ds(s * ZSEG, ZSEG)],
                    out_hbm.at[pl.ds(base_out(s, c), ZSEG)])


def base_out(s, c):
    return c * HALF + s * ZSEG


def _counts_stage(neighbor_mask):
    f = pl.kernel(
        _counts_body,
        out_type=jax.ShapeDtypeStruct((B * A * A,), jnp.float32),
        mesh=plsc.VectorSubcoreMesh(core_axis_name="c", subcore_axis_name="s"),
        scratch_types=[
            pltpu.VMEM((1, CH), jnp.int32),
            pltpu.VMEM((1, CH), jnp.int32),
            pltpu.VMEM((1, CH), jnp.int32),
            pltpu.VMEM((1, CH), jnp.int32),
            pltpu.VMEM((1, CH), jnp.int32),
            pltpu.VMEM((1, CH), jnp.int32),
            pltpu.VMEM((TROWS, 128), jnp.int32),
            pltpu.VMEM((128,), jnp.float32),
            pltpu.VMEM((ZB,), jnp.float32),
            pltpu.VMEM_SHARED((HALF + TPAD,), jnp.float32),
            pltpu.SemaphoreType.DMA,
            pltpu.SemaphoreType.DMA,
            pltpu.SemaphoreType.DMA,
            pltpu.SemaphoreType.DMA,
        ],
    )
    return f(neighbor_mask)


def kernel(positions, neighbor_mask):
    counts = _counts_stage(neighbor_mask).reshape(B, A, A)
    return _dense_stage(positions, counts)
